# Initial kernel scaffold; baseline (speedup 1.0000x reference)
import jax
import jax.numpy as jnp
from jax.experimental import pallas as pl


def kernel(x, emb_table):
    raise NotImplementedError("write your pallas kernel here")


# speedup vs baseline: 1.6634x; 1.6634x over previous
import jax
import jax.numpy as jnp
from jax.experimental import pallas as pl


_BLOCK_T = 1024


def _add_kernel(x_ref, emb_ref, out_ref):
    out_ref[...] = x_ref[...] + emb_ref[...]


def kernel(x, emb_table):
    B, T, D = x.shape
    nt = T // _BLOCK_T
    return pl.pallas_call(
        _add_kernel,
        grid=(nt, B),
        in_specs=[
            pl.BlockSpec((1, _BLOCK_T, D), lambda i, b: (b, i, 0)),
            pl.BlockSpec((_BLOCK_T, D), lambda i, b: (i, 0)),
        ],
        out_specs=pl.BlockSpec((1, _BLOCK_T, D), lambda i, b: (b, i, 0)),
        out_shape=jax.ShapeDtypeStruct((B, T, D), x.dtype),
    )(x, emb_table)
